# baseline (speedup 1.0000x reference)
import jax
import jax.numpy as jnp
from jax import lax
from jax.experimental import pallas as pl
from jax.experimental.pallas import tpu as pltpu
from jax.experimental.pallas import tpu_sc as plsc

NUM_RELATION = 1000
DIM = 64
BATCH = 16384

NC = 2
NS = 16
LANES = 16
NW = NC * NS
BPW = BATCH // NW
GROUPS = BPW // LANES


NCHUNK = 2
CROWS = BPW // NCHUNK
CGROUPS = CROWS // LANES


def _distmult_kernel(sub_hbm, obj_hbm, rela_hbm, diag_hbm, out_hbm,
                     tab_v, sub_v, obj_v, idx_v, out_v,
                     sem_t, sem_s, sem_o, sem_i):
    wid = lax.axis_index("s") * NC + lax.axis_index("c")
    base = wid * BPW

    cp_t = pltpu.make_async_copy(diag_hbm, tab_v, sem_t)
    cp_t.start()
    cp_i = pltpu.make_async_copy(rela_hbm.at[pl.ds(base, BPW)], idx_v, sem_i)
    cp_i.start()

    lane = lax.iota(jnp.int32, LANES)

    for c in range(NCHUNK):
        cbase = (base + c * CROWS) * DIM
        cp_s = pltpu.make_async_copy(
            sub_hbm.at[pl.ds(cbase, CROWS * DIM)], sub_v, sem_s)
        cp_s.start()
        cp_o = pltpu.make_async_copy(
            obj_hbm.at[pl.ds(cbase, CROWS * DIM)], obj_v, sem_o)
        cp_o.start()
        if c == 0:
            cp_t.wait()
            cp_i.wait()
        cp_s.wait()
        cp_o.wait()

        def g_body(g, carry):
            rv = idx_v[pl.ds(c * CROWS + g * LANES, LANES)]
            rbase = rv * DIM
            sbase = g * (LANES * DIM) + lane * DIM
            acc = jnp.zeros((LANES,), jnp.float32)
            for d in range(DIM):
                s = plsc.load_gather(sub_v, [sbase + d])
                dd = plsc.load_gather(tab_v, [rbase + d])
                o = plsc.load_gather(obj_v, [sbase + d])
                acc = acc + s * dd * o
            out_v[pl.ds(c * CROWS + g * LANES, LANES)] = acc
            return carry

        lax.fori_loop(0, CGROUPS, g_body, 0, unroll=False)

    pltpu.sync_copy(out_v, out_hbm.at[pl.ds(base, BPW)])


@jax.jit
def kernel(sub_embed, obj_embed, rela, diag):
    mesh = plsc.VectorSubcoreMesh(core_axis_name="c", subcore_axis_name="s")
    run = pl.kernel(
        _distmult_kernel,
        out_type=jax.ShapeDtypeStruct((BATCH,), jnp.float32),
        mesh=mesh,
        scratch_types=[
            pltpu.VMEM((NUM_RELATION * DIM,), jnp.float32),
            pltpu.VMEM((CROWS * DIM,), jnp.float32),
            pltpu.VMEM((CROWS * DIM,), jnp.float32),
            pltpu.VMEM((BPW,), jnp.int32),
            pltpu.VMEM((BPW,), jnp.float32),
            pltpu.SemaphoreType.DMA,
            pltpu.SemaphoreType.DMA,
            pltpu.SemaphoreType.DMA,
            pltpu.SemaphoreType.DMA,
        ],
        compiler_params=pltpu.CompilerParams(needs_layout_passes=False),
    )
    return run(sub_embed.reshape(-1), obj_embed.reshape(-1),
               rela.astype(jnp.int32), diag.reshape(-1))


# speedup vs baseline: 2.0764x; 2.0764x over previous
import jax
import jax.numpy as jnp
from jax import lax
from jax.experimental import pallas as pl
from jax.experimental.pallas import tpu as pltpu
from jax.experimental.pallas import tpu_sc as plsc

NUM_RELATION = 1000
DIM = 64
BATCH = 16384

NC = 2
NS = 16
LANES = 16
NW = NC * NS
BPW = BATCH // NW
NCHUNK = 4
CROWS = BPW // NCHUNK
CGROUPS = CROWS // LANES


def _distmult_kernel(sub_hbm, obj_hbm, rela_hbm, diag_hbm, out_hbm,
                     tab_v, sub_v, obj_v, idx_v, out_v,
                     sem_t, sem_s, sem_o, sem_i):
    wid = lax.axis_index("s") * NC + lax.axis_index("c")
    base = wid * BPW

    cp_t = pltpu.make_async_copy(diag_hbm, tab_v, sem_t)
    cp_t.start()
    cp_i = pltpu.make_async_copy(rela_hbm.at[pl.ds(base, BPW)], idx_v, sem_i)
    cp_i.start()

    lane = lax.iota(jnp.int32, LANES)

    for c in range(NCHUNK):
        rbase0 = base + c * CROWS
        cp_s = pltpu.make_async_copy(
            sub_hbm.at[pl.ds(rbase0, CROWS)], sub_v, sem_s)
        cp_s.start()
        cp_o = pltpu.make_async_copy(
            obj_hbm.at[pl.ds(rbase0, CROWS)], obj_v, sem_o)
        cp_o.start()
        if c == 0:
            cp_t.wait()
            cp_i.wait()
        cp_s.wait()
        cp_o.wait()

        def g_body(g, carry):
            rv = idx_v[pl.ds(c * CROWS + g * LANES, LANES)]
            row = g * LANES + lane
            acc = jnp.zeros((LANES,), jnp.float32)
            for d in range(DIM):
                col = (lane + d) & (DIM - 1)
                s = plsc.load_gather(sub_v, [row, col])
                dd = plsc.load_gather(tab_v, [rv * DIM + col])
                o = plsc.load_gather(obj_v, [row, col])
                acc = acc + s * dd * o
            out_v[pl.ds(c * CROWS + g * LANES, LANES)] = acc
            return carry

        lax.fori_loop(0, CGROUPS, g_body, 0, unroll=False)

    pltpu.sync_copy(out_v, out_hbm.at[pl.ds(base, BPW)])


@jax.jit
def kernel(sub_embed, obj_embed, rela, diag):
    mesh = plsc.VectorSubcoreMesh(core_axis_name="c", subcore_axis_name="s")
    run = pl.kernel(
        _distmult_kernel,
        out_type=jax.ShapeDtypeStruct((BATCH,), jnp.float32),
        mesh=mesh,
        scratch_types=[
            pltpu.VMEM((NUM_RELATION * DIM,), jnp.float32),
            pltpu.VMEM((CROWS, DIM), jnp.float32),
            pltpu.VMEM((CROWS, DIM), jnp.float32),
            pltpu.VMEM((BPW,), jnp.int32),
            pltpu.VMEM((BPW,), jnp.float32),
            pltpu.SemaphoreType.DMA,
            pltpu.SemaphoreType.DMA,
            pltpu.SemaphoreType.DMA,
            pltpu.SemaphoreType.DMA,
        ],
        compiler_params=pltpu.CompilerParams(needs_layout_passes=False),
    )
    return run(sub_embed, obj_embed, rela.astype(jnp.int32), diag.reshape(-1))


# speedup vs baseline: 2.0804x; 1.0019x over previous
import jax
import jax.numpy as jnp
from jax import lax
from jax.experimental import pallas as pl
from jax.experimental.pallas import tpu as pltpu
from jax.experimental.pallas import tpu_sc as plsc

NUM_RELATION = 1000
DIM = 64
BATCH = 16384

NC = 2
NS = 16
LANES = 16
NW = NC * NS
BPW = BATCH // NW
NCHUNK = 4
CROWS = BPW // NCHUNK
CGROUPS = CROWS // LANES


def _distmult_kernel(sub_hbm, obj_hbm, rela_hbm, diag_hbm, out_hbm,
                     tab_v, sub_v, obj_v, idx_v, out_v,
                     sem_t, sem_s, sem_o, sem_i):
    wid = lax.axis_index("s") * NC + lax.axis_index("c")
    base = wid * BPW

    cp_t = pltpu.make_async_copy(diag_hbm, tab_v, sem_t)
    cp_t.start()
    cp_i = pltpu.make_async_copy(rela_hbm.at[pl.ds(base, BPW)], idx_v, sem_i)
    cp_i.start()

    lane = lax.iota(jnp.int32, LANES)

    for c in range(NCHUNK):
        rbase0 = base + c * CROWS
        cp_s = pltpu.make_async_copy(
            sub_hbm.at[pl.ds(rbase0, CROWS)], sub_v, sem_s)
        cp_s.start()
        cp_o = pltpu.make_async_copy(
            obj_hbm.at[pl.ds(rbase0, CROWS)], obj_v, sem_o)
        cp_o.start()
        if c == 0:
            cp_t.wait()
            cp_i.wait()
        cp_s.wait()
        cp_o.wait()

        def g_body(g, carry):
            rv = idx_v[pl.ds(c * CROWS + g * LANES, LANES)]
            row = g * LANES + lane
            acc = jnp.zeros((LANES,), jnp.float32)
            for d in range(DIM):
                col = (lane + d) & (DIM - 1)
                s = plsc.load_gather(sub_v, [row, col])
                dd = plsc.load_gather(tab_v, [rv * DIM + col])
                o = plsc.load_gather(obj_v, [row, col])
                acc = acc + s * dd * o
            out_v[pl.ds(c * CROWS + g * LANES, LANES)] = acc
            return carry

        lax.fori_loop(0, CGROUPS, g_body, 0, unroll=False)

    pltpu.sync_copy(out_v, out_hbm.at[pl.ds(base, BPW)])


@jax.jit
def kernel(sub_embed, obj_embed, rela, diag):
    mesh = plsc.VectorSubcoreMesh(core_axis_name="c", subcore_axis_name="s")
    run = pl.kernel(
        _distmult_kernel,
        out_type=jax.ShapeDtypeStruct((BATCH,), jnp.float32),
        mesh=mesh,
        scratch_types=[
            pltpu.VMEM((NUM_RELATION * DIM,), jnp.float32),
            pltpu.VMEM((CROWS, DIM), jnp.float32),
            pltpu.VMEM((CROWS, DIM), jnp.float32),
            pltpu.VMEM((BPW,), jnp.int32),
            pltpu.VMEM((BPW,), jnp.float32),
            pltpu.SemaphoreType.DMA,
            pltpu.SemaphoreType.DMA,
            pltpu.SemaphoreType.DMA,
            pltpu.SemaphoreType.DMA,
        ],
        compiler_params=pltpu.CompilerParams(needs_layout_passes=False,
                                             use_tc_tiling_on_sc=True),
    )
    return run(sub_embed, obj_embed, rela.astype(jnp.int32), diag.reshape(-1))


# speedup vs baseline: 2.2640x; 1.0883x over previous
import jax
import jax.numpy as jnp
from jax import lax
from jax.experimental import pallas as pl
from jax.experimental.pallas import tpu as pltpu
from jax.experimental.pallas import tpu_sc as plsc

NUM_RELATION = 1000
DIM = 64
BATCH = 16384

NC = 2
NS = 16
LANES = 16
NW = NC * NS
BPW = BATCH // NW
NCHUNK = 2
CCOLS = BPW // NCHUNK
CGROUPS = CCOLS // LANES


def _distmult_kernel(subT_hbm, objT_hbm, rela_hbm, diag_hbm, out_hbm,
                     tab_v, sub_v, obj_v, idx_v, out_v,
                     sem_t, sem_s, sem_o, sem_i):
    wid = lax.axis_index("s") * NC + lax.axis_index("c")
    base = wid * BPW

    cp_t = pltpu.make_async_copy(diag_hbm, tab_v, sem_t)
    cp_t.start()
    cp_i = pltpu.make_async_copy(rela_hbm.at[pl.ds(base, BPW)], idx_v, sem_i)
    cp_i.start()

    lane = lax.iota(jnp.int32, LANES)

    for c in range(NCHUNK):
        cbase = base + c * CCOLS
        cp_s = pltpu.make_async_copy(
            subT_hbm.at[:, pl.ds(cbase, CCOLS)], sub_v, sem_s)
        cp_s.start()
        cp_o = pltpu.make_async_copy(
            objT_hbm.at[:, pl.ds(cbase, CCOLS)], obj_v, sem_o)
        cp_o.start()
        if c == 0:
            cp_t.wait()
            cp_i.wait()
        cp_s.wait()
        cp_o.wait()

        def g_body(g, carry):
            rv = idx_v[pl.ds(c * CCOLS + g * LANES, LANES)]
            bcol = g * LANES + lane
            acc = jnp.zeros((LANES,), jnp.float32)
            for d in range(DIM):
                drow = (lane + d) & (DIM - 1)
                s = plsc.load_gather(sub_v, [drow, bcol])
                dd = plsc.load_gather(tab_v, [rv * DIM + drow])
                o = plsc.load_gather(obj_v, [drow, bcol])
                acc = acc + s * dd * o
            out_v[pl.ds(c * CCOLS + g * LANES, LANES)] = acc
            return carry

        lax.fori_loop(0, CGROUPS, g_body, 0, unroll=False)

    pltpu.sync_copy(out_v, out_hbm.at[pl.ds(base, BPW)])


@jax.jit
def kernel(sub_embed, obj_embed, rela, diag):
    mesh = plsc.VectorSubcoreMesh(core_axis_name="c", subcore_axis_name="s")
    run = pl.kernel(
        _distmult_kernel,
        out_type=jax.ShapeDtypeStruct((BATCH,), jnp.float32),
        mesh=mesh,
        scratch_types=[
            pltpu.VMEM((NUM_RELATION * DIM,), jnp.float32),
            pltpu.VMEM((DIM, CCOLS), jnp.float32),
            pltpu.VMEM((DIM, CCOLS), jnp.float32),
            pltpu.VMEM((BPW,), jnp.int32),
            pltpu.VMEM((BPW,), jnp.float32),
            pltpu.SemaphoreType.DMA,
            pltpu.SemaphoreType.DMA,
            pltpu.SemaphoreType.DMA,
            pltpu.SemaphoreType.DMA,
        ],
        compiler_params=pltpu.CompilerParams(needs_layout_passes=False),
    )
    return run(jnp.swapaxes(sub_embed, 0, 1), jnp.swapaxes(obj_embed, 0, 1),
               rela.astype(jnp.int32), diag.reshape(-1))
